# baseline (speedup 1.0000x reference)
import jax
import jax.numpy as jnp
from jax import lax
from jax.experimental import pallas as pl
from jax.experimental.pallas import tpu as pltpu
from jax.experimental.pallas import tpu_sc as plsc

_TOPK = 64
_N = 2048
_F = 2048
_NC, _NS, _L = 2, 16, 16
_NW = _NC * _NS
_RPW = _N // _NW
_NB = _RPW // _L
_NBINS = 256
_HSTRIDE = _NBINS + 1
_CSTRIDE = _F + 9
_MANT_HI = 0x800000
_ABS_MASK = 0x7FFFFFFF


def _sc_body_fn(rpw, nb):
  def _sc_body(y_hbm, thr_hbm, in_v, cand_v, cand2_v, hist_v, thr_v, sem):
    del sem
    wid = lax.axis_index("s") * _NC + lax.axis_index("c")
    lanes = lax.iota(jnp.int32, 16)
    ones = jnp.ones((_L,), jnp.int32)
    zeros16 = jnp.zeros((_L,), jnp.int32)
    lhist = lanes * _HSTRIDE
    lcand = lanes * _CSTRIDE
    lh16 = lanes * 17
    row0 = wid * rpw

    @plsc.parallel_loop(0, _L * _HSTRIDE // _L + 1, unroll=4)
    def _(b):
        hist_v[pl.ds(b * _L, _L)] = zeros16

    def do_batch(bi, _):
        pltpu.sync_copy(y_hbm.at[pl.ds(row0 + bi * _L, _L)], in_v)

        @plsc.parallel_loop(0, _F, unroll=4)
        def _(j):
            jr = (j + lanes) & (_F - 1)
            col = plsc.load_gather(in_v, [lanes, jr])
            bits = plsc.bitcast(col, jnp.int32) & _ABS_MASK
            plsc.addupdate_scatter(hist_v, [lhist + (bits >> 23)], ones)

        def scan_bin(k, carry):
            cum, estar, mneed = carry
            b = (_NBINS - 1) - k
            idx = lhist + b
            h = plsc.load_gather(hist_v, [idx])
            plsc.store_scatter(hist_v, [idx], zeros16)
            newcum = cum + h
            hit = (cum < _TOPK) & (newcum >= _TOPK)
            estar = jnp.where(hit, b, estar)
            mneed = jnp.where(hit, _TOPK - cum, mneed)
            return newcum, estar, mneed
        _cum, estar, mneed = lax.fori_loop(
            0, _NBINS, scan_bin, (zeros16, zeros16, ones), unroll=2)

        @plsc.parallel_loop(0, _F, unroll=4, carry=zeros16)
        def nbe(j, off):
            jr = (j + lanes) & (_F - 1)
            col = plsc.load_gather(in_v, [lanes, jr])
            bits = plsc.bitcast(col, jnp.int32) & _ABS_MASK
            msk = (bits >> 23) == estar
            plsc.store_scatter(cand_v, [lcand + off], bits & (_MANT_HI - 1),
                               mask=msk)
            return off + jnp.where(msk, 1, 0)
        maxn = jnp.max(nbe)

        def refine(shift, src, nsrc, maxnsrc, need, dst):
            @plsc.parallel_loop(0, maxnsrc)
            def _(i):
                c = plsc.load_gather(src, [lcand + i])
                d = (c >> shift) & 15
                plsc.addupdate_scatter(hist_v, [lh16 + d],
                                       jnp.where(i < nsrc, 1, 0))

            def scan16(k, carry):
                cum, dstar, mrem = carry
                b = 15 - k
                idx = lh16 + b
                h = plsc.load_gather(hist_v, [idx])
                plsc.store_scatter(hist_v, [idx], zeros16)
                newcum = cum + h
                hit = (cum < need) & (newcum >= need)
                dstar = jnp.where(hit, b, dstar)
                mrem = jnp.where(hit, need - cum, mrem)
                return newcum, dstar, mrem
            _c, dstar, mrem = lax.fori_loop(
                0, 16, scan16, (zeros16, zeros16, ones))

            @plsc.parallel_loop(0, maxnsrc, carry=zeros16)
            def ndst(i, off):
                c = plsc.load_gather(src, [lcand + i])
                msk = (((c >> shift) & 15) == dstar) & (i < nsrc)
                plsc.store_scatter(dst, [lcand + off], c, mask=msk)
                return off + jnp.where(msk, 1, 0)
            return dstar, mrem, ndst, jnp.max(ndst)

        d1, m1, n2, maxn2 = refine(19, cand_v, nbe, maxn, mneed, cand2_v)
        d2, m2, n3, maxn3 = refine(15, cand2_v, n2, maxn2, m1, cand_v)

        prefix = (d1 << 19) + (d2 << 15)

        def bis(_, carry):
            lo, hi = carry
            mid = lo + ((hi - lo) >> 1)

            @plsc.parallel_loop(0, maxn3, carry=zeros16)
            def cnt(i, acc):
                c = plsc.load_gather(cand_v, [lcand + i])
                good = (i < n3) & (c >= mid)
                return acc + jnp.where(good, 1, 0)
            ge = cnt >= m2
            return jnp.where(ge, mid, lo), jnp.where(ge, hi, mid)
        lo, _hi = lax.fori_loop(0, 15, bis, (prefix, prefix + (1 << 15)))
        thr_v[pl.ds(bi * _L, _L)] = (estar << 23) + lo
        return 0

    lax.fori_loop(0, nb, do_batch, 0)
    pltpu.sync_copy(thr_v, thr_hbm.at[pl.ds(row0, rpw)])
  return _sc_body


def _sc_thresholds(y, nrows):
    rpw = nrows // _NW
    nb = rpw // _L
    mesh = plsc.VectorSubcoreMesh(core_axis_name="c", subcore_axis_name="s")
    return pl.kernel(
        _sc_body_fn(rpw, nb),
        out_type=jax.ShapeDtypeStruct((nrows,), jnp.int32),
        mesh=mesh,
        compiler_params=pltpu.CompilerParams(needs_layout_passes=False),
        scratch_types=[
            pltpu.VMEM((_L, _F), jnp.float32),
            pltpu.VMEM((_L * _CSTRIDE,), jnp.int32),
            pltpu.VMEM((_L * _CSTRIDE,), jnp.int32),
            pltpu.VMEM((_L * _HSTRIDE + _L,), jnp.int32),
            pltpu.VMEM((rpw,), jnp.int32),
            pltpu.SemaphoreType.DMA,
        ],
    )(y)


def _mm_body(x_ref, w_ref, b_ref, o_ref):
    o_ref[...] = jax.lax.dot_general(
        x_ref[...], w_ref[...],
        dimension_numbers=(((1,), (1,)), ((), ())),
        preferred_element_type=jnp.float32,
    ) + b_ref[...]


def _matmul(x, weight, bias):
    n, fin = x.shape
    fout = weight.shape[0]
    br = 256
    return pl.pallas_call(
        _mm_body,
        grid=(n // br,),
        in_specs=[
            pl.BlockSpec((br, fin), lambda i: (i, 0)),
            pl.BlockSpec((fout, fin), lambda i: (0, 0)),
            pl.BlockSpec((1, fout), lambda i: (0, 0)),
        ],
        out_specs=pl.BlockSpec((br, fout), lambda i: (i, 0)),
        out_shape=jax.ShapeDtypeStruct((n, fout), jnp.float32),
    )(x, weight, bias.reshape(1, fout))


def _mask_body(y_ref, t_ref, o_ref):
    y = y_ref[...]
    bits = jax.lax.bitcast_convert_type(jnp.abs(y), jnp.int32)
    o_ref[...] = jnp.where(bits >= t_ref[...], y, 0.0)


def _mask(y, thr, nrows):
    br = 256
    return pl.pallas_call(
        _mask_body,
        grid=(nrows // br,),
        in_specs=[
            pl.BlockSpec((br, _F), lambda i: (i, 0)),
            pl.BlockSpec((br, 1), lambda i: (i, 0)),
        ],
        out_specs=pl.BlockSpec((br, _F), lambda i: (i, 0)),
        out_shape=jax.ShapeDtypeStruct((nrows, _F), jnp.float32),
    )(y, thr.reshape(nrows, 1))


_CHUNKS = 2


def kernel(x, weight, bias):
    ch = _N // _CHUNKS
    outs = []
    for c in range(_CHUNKS):
        xc = lax.slice_in_dim(x, c * ch, (c + 1) * ch, axis=0)
        yc = _matmul(xc, weight, bias)
        thr = _sc_thresholds(yc, ch)
        outs.append(_mask(yc, thr, ch))
    return jnp.concatenate(outs, axis=0)


# speedup vs baseline: 1.1425x; 1.1425x over previous
import jax
import jax.numpy as jnp
from jax import lax
from jax.experimental import pallas as pl
from jax.experimental.pallas import tpu as pltpu
from jax.experimental.pallas import tpu_sc as plsc

_TOPK = 64
_N = 2048
_F = 2048
_NC, _NS, _L = 2, 16, 16
_NW = _NC * _NS
_RPW = _N // _NW
_NB = _RPW // _L
_NBINS = 256
_HSTRIDE = _NBINS + 1
_CSTRIDE = _F + 9
_MANT_HI = 0x800000
_ABS_MASK = 0x7FFFFFFF


def _sc_body(y_hbm, thr_hbm, in_v, cand_v, cand2_v, hist_v, thr_v, sem):
    del sem
    wid = lax.axis_index("s") * _NC + lax.axis_index("c")
    lanes = lax.iota(jnp.int32, 16)
    ones = jnp.ones((_L,), jnp.int32)
    zeros16 = jnp.zeros((_L,), jnp.int32)
    lhist = lanes * _HSTRIDE
    lcand = lanes * _CSTRIDE
    lh16 = lanes * 17
    row0 = wid * _RPW

    @plsc.parallel_loop(0, _L * _HSTRIDE // _L + 1, unroll=4)
    def _(b):
        hist_v[pl.ds(b * _L, _L)] = zeros16

    def do_batch(bi, _):
        pltpu.sync_copy(y_hbm.at[pl.ds(row0 + bi * _L, _L)], in_v)

        @plsc.parallel_loop(0, _F, unroll=8)
        def _(j):
            jr = (j + lanes) & (_F - 1)
            col = plsc.load_gather(in_v, [lanes, jr])
            bits = plsc.bitcast(col, jnp.int32) & _ABS_MASK
            plsc.addupdate_scatter(hist_v, [lhist + (bits >> 23)], ones)

        def scan_bin(k, carry):
            cum, estar, mneed = carry
            b = (_NBINS - 1) - k
            idx = lhist + b
            h = plsc.load_gather(hist_v, [idx])
            plsc.store_scatter(hist_v, [idx], zeros16)
            newcum = cum + h
            hit = (cum < _TOPK) & (newcum >= _TOPK)
            estar = jnp.where(hit, b, estar)
            mneed = jnp.where(hit, _TOPK - cum, mneed)
            return newcum, estar, mneed
        _cum, estar, mneed = lax.fori_loop(
            0, _NBINS, scan_bin, (zeros16, zeros16, ones), unroll=4)

        @plsc.parallel_loop(0, _F, unroll=8, carry=zeros16)
        def nbe(j, off):
            jr = (j + lanes) & (_F - 1)
            col = plsc.load_gather(in_v, [lanes, jr])
            bits = plsc.bitcast(col, jnp.int32) & _ABS_MASK
            msk = (bits >> 23) == estar
            plsc.store_scatter(cand_v, [lcand + off], bits & (_MANT_HI - 1),
                               mask=msk)
            return off + jnp.where(msk, 1, 0)
        maxn = jnp.max(nbe)

        def refine(shift, src, nsrc, maxnsrc, need, dst):
            @plsc.parallel_loop(0, maxnsrc)
            def _(i):
                c = plsc.load_gather(src, [lcand + i])
                d = (c >> shift) & 15
                plsc.addupdate_scatter(hist_v, [lh16 + d],
                                       jnp.where(i < nsrc, 1, 0))

            def scan16(k, carry):
                cum, dstar, mrem = carry
                b = 15 - k
                idx = lh16 + b
                h = plsc.load_gather(hist_v, [idx])
                plsc.store_scatter(hist_v, [idx], zeros16)
                newcum = cum + h
                hit = (cum < need) & (newcum >= need)
                dstar = jnp.where(hit, b, dstar)
                mrem = jnp.where(hit, need - cum, mrem)
                return newcum, dstar, mrem
            _c, dstar, mrem = lax.fori_loop(
                0, 16, scan16, (zeros16, zeros16, ones))

            @plsc.parallel_loop(0, maxnsrc, carry=zeros16)
            def ndst(i, off):
                c = plsc.load_gather(src, [lcand + i])
                msk = (((c >> shift) & 15) == dstar) & (i < nsrc)
                plsc.store_scatter(dst, [lcand + off], c, mask=msk)
                return off + jnp.where(msk, 1, 0)
            return dstar, mrem, ndst, jnp.max(ndst)

        d1, m1, n2, maxn2 = refine(19, cand_v, nbe, maxn, mneed, cand2_v)
        d2, m2, n3, maxn3 = refine(15, cand2_v, n2, maxn2, m1, cand_v)

        prefix = (d1 << 19) + (d2 << 15)

        def bis(_, carry):
            lo, hi = carry
            mid = lo + ((hi - lo) >> 1)

            @plsc.parallel_loop(0, maxn3, carry=zeros16)
            def cnt(i, acc):
                c = plsc.load_gather(cand_v, [lcand + i])
                good = (i < n3) & (c >= mid)
                return acc + jnp.where(good, 1, 0)
            ge = cnt >= m2
            return jnp.where(ge, mid, lo), jnp.where(ge, hi, mid)
        lo, _hi = lax.fori_loop(0, 15, bis, (prefix, prefix + (1 << 15)))
        thr_v[pl.ds(bi * _L, _L)] = (estar << 23) + lo
        return 0

    lax.fori_loop(0, _NB, do_batch, 0)
    pltpu.sync_copy(thr_v, thr_hbm.at[pl.ds(row0, _RPW)])


def _sc_thresholds(y):
    mesh = plsc.VectorSubcoreMesh(core_axis_name="c", subcore_axis_name="s")
    return pl.kernel(
        _sc_body,
        out_type=jax.ShapeDtypeStruct((_N,), jnp.int32),
        mesh=mesh,
        compiler_params=pltpu.CompilerParams(needs_layout_passes=False),
        scratch_types=[
            pltpu.VMEM((_L, _F), jnp.float32),
            pltpu.VMEM((_L * _CSTRIDE,), jnp.int32),
            pltpu.VMEM((_L * _CSTRIDE,), jnp.int32),
            pltpu.VMEM((_L * _HSTRIDE + _L,), jnp.int32),
            pltpu.VMEM((_RPW,), jnp.int32),
            pltpu.SemaphoreType.DMA,
        ],
    )(y)


def _mm_body(x_ref, w_ref, b_ref, o_ref):
    o_ref[...] = jax.lax.dot_general(
        x_ref[...], w_ref[...],
        dimension_numbers=(((1,), (1,)), ((), ())),
        preferred_element_type=jnp.float32,
    ) + b_ref[...]


def _matmul(x, weight, bias):
    n, fin = x.shape
    fout = weight.shape[0]
    br = 256
    return pl.pallas_call(
        _mm_body,
        grid=(n // br,),
        in_specs=[
            pl.BlockSpec((br, fin), lambda i: (i, 0)),
            pl.BlockSpec((fout, fin), lambda i: (0, 0)),
            pl.BlockSpec((1, fout), lambda i: (0, 0)),
        ],
        out_specs=pl.BlockSpec((br, fout), lambda i: (i, 0)),
        out_shape=jax.ShapeDtypeStruct((n, fout), jnp.float32),
    )(x, weight, bias.reshape(1, fout))


def _mask_body(y_ref, t_ref, o_ref):
    y = y_ref[...]
    bits = jax.lax.bitcast_convert_type(jnp.abs(y), jnp.int32)
    o_ref[...] = jnp.where(bits >= t_ref[...], y, 0.0)


def _mask(y, thr):
    br = 256
    return pl.pallas_call(
        _mask_body,
        grid=(_N // br,),
        in_specs=[
            pl.BlockSpec((br, _F), lambda i: (i, 0)),
            pl.BlockSpec((br, 1), lambda i: (i, 0)),
        ],
        out_specs=pl.BlockSpec((br, _F), lambda i: (i, 0)),
        out_shape=jax.ShapeDtypeStruct((_N, _F), jnp.float32),
    )(y, thr.reshape(_N, 1))


def kernel(x, weight, bias):
    y = _matmul(x, weight, bias)
    thr = _sc_thresholds(y)
    return _mask(y, thr)


# speedup vs baseline: 1.1601x; 1.0154x over previous
import jax
import jax.numpy as jnp
from jax import lax
from jax.experimental import pallas as pl
from jax.experimental.pallas import tpu as pltpu
from jax.experimental.pallas import tpu_sc as plsc

_TOPK = 64
_N = 2048
_F = 2048
_NC, _NS, _L = 2, 16, 16
_NW = _NC * _NS
_RPW = _N // _NW
_NB = _RPW // _L
_NBINS = 256
_HSTRIDE = _NBINS + 1
_CSTRIDE = _F + 9
_MANT_HI = 0x800000
_ABS_MASK = 0x7FFFFFFF


def _sc_body(y_hbm, thr_hbm, in_v, cand_v, cand2_v, hist_v, thr_v, sem):
    del sem
    wid = lax.axis_index("s") * _NC + lax.axis_index("c")
    lanes = lax.iota(jnp.int32, 16)
    ones = jnp.ones((_L,), jnp.int32)
    zeros16 = jnp.zeros((_L,), jnp.int32)
    lhist = lanes * _HSTRIDE
    lcand = lanes * _CSTRIDE
    lh16 = lanes * 17
    row0 = wid * _RPW

    @plsc.parallel_loop(0, _L * _HSTRIDE // _L + 1, unroll=4)
    def _(b):
        hist_v[pl.ds(b * _L, _L)] = zeros16

    def do_batch(bi, _):
        pltpu.sync_copy(y_hbm.at[pl.ds(row0 + bi * _L, _L)], in_v)

        @plsc.parallel_loop(0, _F, unroll=8)
        def _(j):
            jr = (j + lanes) & (_F - 1)
            col = plsc.load_gather(in_v, [lanes, jr])
            bits = plsc.bitcast(col, jnp.int32) & _ABS_MASK
            plsc.addupdate_scatter(hist_v, [lhist + (bits >> 23)], ones)

        def scan_bin(k, carry):
            cum, estar, mneed = carry
            b = (_NBINS - 1) - k
            idx = lhist + b
            h = plsc.load_gather(hist_v, [idx])
            plsc.store_scatter(hist_v, [idx], zeros16)
            newcum = cum + h
            hit = (cum < _TOPK) & (newcum >= _TOPK)
            estar = jnp.where(hit, b, estar)
            mneed = jnp.where(hit, _TOPK - cum, mneed)
            return newcum, estar, mneed
        _cum, estar, mneed = lax.fori_loop(
            0, _NBINS, scan_bin, (zeros16, zeros16, ones), unroll=4)

        @plsc.parallel_loop(0, _F, unroll=8, carry=zeros16)
        def nbe(j, off):
            jr = (j + lanes) & (_F - 1)
            col = plsc.load_gather(in_v, [lanes, jr])
            bits = plsc.bitcast(col, jnp.int32) & _ABS_MASK
            msk = (bits >> 23) == estar
            plsc.store_scatter(cand_v, [lcand + off], bits & (_MANT_HI - 1),
                               mask=msk)
            return off + jnp.where(msk, 1, 0)
        maxn = jnp.max(nbe)

        def refine(shift, src, nsrc, maxnsrc, need, dst):
            @plsc.parallel_loop(0, maxnsrc)
            def _(i):
                c = plsc.load_gather(src, [lcand + i])
                d = (c >> shift) & 15
                plsc.addupdate_scatter(hist_v, [lh16 + d],
                                       jnp.where(i < nsrc, 1, 0))

            def scan16(k, carry):
                cum, dstar, mrem = carry
                b = 15 - k
                idx = lh16 + b
                h = plsc.load_gather(hist_v, [idx])
                plsc.store_scatter(hist_v, [idx], zeros16)
                newcum = cum + h
                hit = (cum < need) & (newcum >= need)
                dstar = jnp.where(hit, b, dstar)
                mrem = jnp.where(hit, need - cum, mrem)
                return newcum, dstar, mrem
            _c, dstar, mrem = lax.fori_loop(
                0, 16, scan16, (zeros16, zeros16, ones))

            @plsc.parallel_loop(0, maxnsrc, carry=zeros16)
            def ndst(i, off):
                c = plsc.load_gather(src, [lcand + i])
                msk = (((c >> shift) & 15) == dstar) & (i < nsrc)
                plsc.store_scatter(dst, [lcand + off], c, mask=msk)
                return off + jnp.where(msk, 1, 0)
            return dstar, mrem, ndst, jnp.max(ndst)

        d1, m1, n2, maxn2 = refine(19, cand_v, nbe, maxn, mneed, cand2_v)
        d2, m2, n3, maxn3 = refine(15, cand2_v, n2, maxn2, m1, cand_v)

        prefix = (d1 << 19) + (d2 << 15)

        def bis(_, carry):
            lo, hi = carry
            mid = lo + ((hi - lo) >> 1)

            @plsc.parallel_loop(0, maxn3, carry=zeros16)
            def cnt(i, acc):
                c = plsc.load_gather(cand_v, [lcand + i])
                good = (i < n3) & (c >= mid)
                return acc + jnp.where(good, 1, 0)
            ge = cnt >= m2
            return jnp.where(ge, mid, lo), jnp.where(ge, hi, mid)
        lo, _hi = lax.fori_loop(0, 15, bis, (prefix, prefix + (1 << 15)))
        thr_v[pl.ds(bi * _L, _L)] = (estar << 23) + lo
        return 0

    lax.fori_loop(0, _NB, do_batch, 0)
    pltpu.sync_copy(thr_v, thr_hbm.at[pl.ds(row0, _RPW)])


def _sc_thresholds(y):
    mesh = plsc.VectorSubcoreMesh(core_axis_name="c", subcore_axis_name="s")
    return pl.kernel(
        _sc_body,
        out_type=jax.ShapeDtypeStruct((_N,), jnp.int32),
        mesh=mesh,
        compiler_params=pltpu.CompilerParams(needs_layout_passes=False),
        scratch_types=[
            pltpu.VMEM((_L, _F), jnp.float32),
            pltpu.VMEM((_L * _CSTRIDE,), jnp.int32),
            pltpu.VMEM((_L * _CSTRIDE,), jnp.int32),
            pltpu.VMEM((_L * _HSTRIDE + _L,), jnp.int32),
            pltpu.VMEM((_RPW,), jnp.int32),
            pltpu.SemaphoreType.DMA,
        ],
    )(y)


def _mm_body(x_ref, w_ref, b_ref, o_ref):
    o_ref[...] = jax.lax.dot_general(
        x_ref[...], w_ref[...],
        dimension_numbers=(((1,), (1,)), ((), ())),
        preferred_element_type=jnp.float32,
    ) + b_ref[...]


def _matmul(x, weight, bias):
    n, fin = x.shape
    fout = weight.shape[0]
    br = 512
    return pl.pallas_call(
        _mm_body,
        grid=(n // br,),
        in_specs=[
            pl.BlockSpec((br, fin), lambda i: (i, 0)),
            pl.BlockSpec((fout, fin), lambda i: (0, 0)),
            pl.BlockSpec((1, fout), lambda i: (0, 0)),
        ],
        out_specs=pl.BlockSpec((br, fout), lambda i: (i, 0)),
        out_shape=jax.ShapeDtypeStruct((n, fout), jnp.float32),
    )(x, weight, bias.reshape(1, fout))


def _mask_body(y_ref, t_ref, o_ref):
    y = y_ref[...]
    bits = jax.lax.bitcast_convert_type(jnp.abs(y), jnp.int32)
    o_ref[...] = jnp.where(bits >= t_ref[...], y, 0.0)


def _mask(y, thr):
    br = 512
    return pl.pallas_call(
        _mask_body,
        grid=(_N // br,),
        in_specs=[
            pl.BlockSpec((br, _F), lambda i: (i, 0)),
            pl.BlockSpec((br, 1), lambda i: (i, 0)),
        ],
        out_specs=pl.BlockSpec((br, _F), lambda i: (i, 0)),
        out_shape=jax.ShapeDtypeStruct((_N, _F), jnp.float32),
    )(y, thr.reshape(_N, 1))


def kernel(x, weight, bias):
    y = _matmul(x, weight, bias)
    thr = _sc_thresholds(y)
    return _mask(y, thr)
